# Initial kernel scaffold; baseline (speedup 1.0000x reference)
import jax
import jax.numpy as jnp
from jax.experimental import pallas as pl


def kernel(T, pos_emb):
    raise NotImplementedError("write your pallas kernel here")


# speedup vs baseline: 1.5785x; 1.5785x over previous
import functools

import jax
import jax.numpy as jnp
from jax import lax
from jax.experimental import pallas as pl
from jax.experimental.pallas import tpu as pltpu
from jax.experimental.pallas import tpu_sc as plsc

_MAX_LEN = 8192
_D = 1024
_NC = 2
_NS = 16
_NW = _NC * _NS
_ROWS_PER_W = _MAX_LEN // _NW
_CHUNK = 32
_NCHUNK = _ROWS_PER_W // _CHUNK
_NBUF = 3


def _sc_gather(table, idx):
    mesh = plsc.VectorSubcoreMesh(
        core_axis_name="c", subcore_axis_name="s",
        num_cores=_NC, num_subcores=_NS)

    @functools.partial(
        pl.kernel,
        out_type=jax.ShapeDtypeStruct((_MAX_LEN, _D), jnp.float32),
        mesh=mesh,
        scratch_types=(
            [pltpu.VMEM((_ROWS_PER_W,), jnp.int32)]
            + [pltpu.VMEM((_CHUNK, _D), jnp.float32) for _ in range(_NBUF)]
            + [pltpu.SemaphoreType.DMA for _ in range(2 * _NBUF)]
        ),
    )
    def k(table_hbm, idx_hbm, out_hbm, idx_v, *rest):
        bufs = rest[:_NBUF]
        gsems = rest[_NBUF:2 * _NBUF]
        wsems = rest[2 * _NBUF:]
        wid = lax.axis_index("s") * _NC + lax.axis_index("c")
        base = wid * _ROWS_PER_W
        pltpu.sync_copy(idx_hbm.at[pl.ds(base, _ROWS_PER_W)], idx_v)

        def gather(c, s):
            return pltpu.async_copy(
                table_hbm.at[idx_v.at[pl.ds(c * _CHUNK, _CHUNK)]],
                bufs[s], gsems[s])

        def put(c, s):
            return pltpu.async_copy(
                bufs[s], out_hbm.at[pl.ds(base + c * _CHUNK, _CHUNK)],
                wsems[s])

        g = [None] * _NCHUNK
        w = [None] * _NCHUNK
        for c in range(min(_NBUF, _NCHUNK)):
            g[c] = gather(c, c % _NBUF)
        for c in range(_NCHUNK):
            s = c % _NBUF
            g[c].wait()
            w[c] = put(c, s)
            n = c + _NBUF
            if n < _NCHUNK:
                w[c].wait()
                g[n] = gather(n, s)
        for c in range(max(0, _NCHUNK - _NBUF), _NCHUNK):
            w[c].wait()

    return k(table, idx)


def kernel(T, pos_emb):
    pos = jnp.arange(_MAX_LEN, dtype=jnp.int32) + (
        jnp.asarray(T, jnp.int32) - _MAX_LEN)
    pos = jnp.clip(pos, 0, _MAX_LEN - 1)
    out = _sc_gather(pos_emb, pos)
    return out[None, :, :]


# speedup vs baseline: 1.5986x; 1.0128x over previous
import functools

import jax
import jax.numpy as jnp
from jax import lax
from jax.experimental import pallas as pl
from jax.experimental.pallas import tpu as pltpu
from jax.experimental.pallas import tpu_sc as plsc

_MAX_LEN = 8192
_D = 1024
_NC = 2
_NS = 16
_NW = _NC * _NS
_ROWS_PER_W = _MAX_LEN // _NW
_CHUNK = 16
_NCHUNK = _ROWS_PER_W // _CHUNK
_NBUF = 7


def _sc_gather(table, idx):
    mesh = plsc.VectorSubcoreMesh(
        core_axis_name="c", subcore_axis_name="s",
        num_cores=_NC, num_subcores=_NS)

    @functools.partial(
        pl.kernel,
        out_type=jax.ShapeDtypeStruct((_MAX_LEN, _D), jnp.float32),
        mesh=mesh,
        scratch_types=(
            [pltpu.VMEM((_ROWS_PER_W,), jnp.int32)]
            + [pltpu.VMEM((_CHUNK, _D), jnp.float32) for _ in range(_NBUF)]
            + [pltpu.SemaphoreType.DMA for _ in range(2 * _NBUF)]
        ),
    )
    def k(table_hbm, idx_hbm, out_hbm, idx_v, *rest):
        bufs = rest[:_NBUF]
        gsems = rest[_NBUF:2 * _NBUF]
        wsems = rest[2 * _NBUF:]
        wid = lax.axis_index("s") * _NC + lax.axis_index("c")
        base = wid * _ROWS_PER_W
        pltpu.sync_copy(idx_hbm.at[pl.ds(base, _ROWS_PER_W)], idx_v)

        def gather(c, s):
            return pltpu.async_copy(
                table_hbm.at[idx_v.at[pl.ds(c * _CHUNK, _CHUNK)]],
                bufs[s], gsems[s])

        def put(c, s):
            return pltpu.async_copy(
                bufs[s], out_hbm.at[pl.ds(base + c * _CHUNK, _CHUNK)],
                wsems[s])

        lead = _NBUF - 1
        g = [None] * _NCHUNK
        w = [None] * _NCHUNK
        unwaited = set()
        for c in range(min(lead, _NCHUNK)):
            g[c] = gather(c, c % _NBUF)
        for c in range(_NCHUNK):
            g[c].wait()
            w[c] = put(c, c % _NBUF)
            unwaited.add(c)
            n = c + lead
            if n < _NCHUNK:
                if c >= 1:
                    w[c - 1].wait()
                    unwaited.discard(c - 1)
                g[n] = gather(n, n % _NBUF)
        for c in sorted(unwaited):
            w[c].wait()

    return k(table, idx)


def kernel(T, pos_emb):
    pos = jnp.arange(_MAX_LEN, dtype=jnp.int32) + (
        jnp.asarray(T, jnp.int32) - _MAX_LEN)
    pos = jnp.clip(pos, 0, _MAX_LEN - 1)
    out = _sc_gather(pos_emb, pos)
    return out[None, :, :]
